# baseline (speedup 1.0000x reference)
import jax
import jax.numpy as jnp
from jax import lax
from jax.experimental import pallas as pl
from jax.experimental.pallas import tpu as pltpu
from jax.experimental.pallas import tpu_sc as plsc

B, T, D = 8, 512, 512
MEL_MAX = 4096

NC, NS = 2, 16
NW = NC * NS
FRAMES_PER_W = B * MEL_MAX // NW
CHUNK = 64
NCHUNK = FRAMES_PER_W // CHUNK
GROUPS = FRAMES_PER_W // 16
ZERO_ROW = B * T


def _lr_body(x_hbm, tgt_hbm, out_hbm, tgt_v, cums_v, gidx_v, buf0, buf1, sem0, sem1):
    cid = lax.axis_index("c")
    sid = lax.axis_index("s")
    wid = sid * NC + cid
    b = wid // (MEL_MAX // FRAMES_PER_W)
    fb = (wid % (MEL_MAX // FRAMES_PER_W)) * FRAMES_PER_W
    row_base = wid * FRAMES_PER_W

    pltpu.sync_copy(tgt_hbm.at[b], tgt_v)

    def cum_step(i, carry):
        v = tgt_v[pl.ds(i * 16, 16)]
        cums_v[pl.ds(i * 16, 16)] = plsc.cumsum(v) + carry
        return carry + jnp.sum(v)

    total = lax.fori_loop(0, T // 16, cum_step, jnp.int32(0))

    lane = lax.iota(jnp.int32, 16)

    def bs_step(gi, _):
        j = fb + gi * 16 + lane
        idx = jnp.zeros((16,), jnp.int32)
        for s in (256, 128, 64, 32, 16, 8, 4, 2, 1):
            val = plsc.load_gather(cums_v, [idx + (s - 1)])
            idx = jnp.where(val <= j, idx + s, idx)
        val = plsc.load_gather(cums_v, [idx])
        cnt = idx + (val <= j).astype(jnp.int32)
        cnt = jnp.minimum(cnt, T - 1)
        g = jnp.where(j < total, b * T + cnt, ZERO_ROW + (j & 63))
        gidx_v[pl.ds(gi * 16, 16)] = g
        return 0

    lax.fori_loop(0, GROUPS, bs_step, 0)

    bufs = (buf0, buf1)
    sems = (sem0, sem1)
    def gidx_slice(c):
        return gidx_v.at[pl.ds(c * CHUNK, CHUNK)]

    cps = [pltpu.async_copy(x_hbm.at[gidx_slice(0)], bufs[0], sems[0]),
           pltpu.async_copy(x_hbm.at[gidx_slice(1)], bufs[1], sems[1])]
    for c in range(NCHUNK):
        k = c % 2
        cps[k].wait()
        pltpu.sync_copy(bufs[k], out_hbm.at[pl.ds(row_base + c * CHUNK, CHUNK)])
        if c + 2 < NCHUNK:
            cps[k] = pltpu.async_copy(x_hbm.at[gidx_slice(c + 2)], bufs[k], sems[k])


def _length_regulate(x_pad, target):
    mesh = plsc.VectorSubcoreMesh(
        core_axis_name="c", subcore_axis_name="s", num_cores=NC, num_subcores=NS)
    k = pl.kernel(
        _lr_body,
        out_type=jax.ShapeDtypeStruct((B * MEL_MAX, D), jnp.float32),
        mesh=mesh,
        compiler_params=pltpu.CompilerParams(needs_layout_passes=False),
        scratch_types=[
            pltpu.VMEM((T,), jnp.int32),
            pltpu.VMEM((T,), jnp.int32),
            pltpu.VMEM((FRAMES_PER_W,), jnp.int32),
            pltpu.VMEM((CHUNK, D), jnp.float32),
            pltpu.VMEM((CHUNK, D), jnp.float32),
            pltpu.SemaphoreType.DMA,
            pltpu.SemaphoreType.DMA,
        ],
    )
    return k(x_pad, target)


def _dp_body(x_ref, w1_ref, b1_ref, w2_ref, b2_ref, o_ref):
    xb = x_ref[0]
    h = jnp.dot(xb, w1_ref[...], preferred_element_type=jnp.float32) + b1_ref[...]
    h = jnp.maximum(h, 0.0)
    d = jnp.sum(h * w2_ref[...], axis=1, keepdims=True) + b2_ref[0, 0]
    d = jnp.maximum(d, 0.0)
    o_ref[...] = jnp.broadcast_to(d, (T, 128))


def _duration_predictor(x, W1, b1, W2, b2):
    out = pl.pallas_call(
        _dp_body,
        grid=(B,),
        in_specs=[
            pl.BlockSpec((1, T, D), lambda i: (i, 0, 0)),
            pl.BlockSpec((D, D), lambda i: (0, 0)),
            pl.BlockSpec((1, D), lambda i: (0, 0)),
            pl.BlockSpec((1, D), lambda i: (0, 0)),
            pl.BlockSpec((1, 1), lambda i: (0, 0)),
        ],
        out_specs=pl.BlockSpec((T, 128), lambda i: (i, 0)),
        out_shape=jax.ShapeDtypeStruct((B * T, 128), jnp.float32),
    )(x, W1, b1.reshape(1, D), W2.reshape(1, D), b2.reshape(1, 1))
    return out[:, 0].reshape(B, T)


def kernel(x, target, mel_max_length, W1, b1, W2, b2):
    del mel_max_length
    dp = _duration_predictor(x, W1, b1, W2, b2)
    x_pad = jnp.concatenate(
        [x.reshape(B * T, D), jnp.zeros((64, D), x.dtype)], axis=0)
    out = _length_regulate(x_pad, target).reshape(B, MEL_MAX, D)
    return out, dp


# speedup vs baseline: 1.4736x; 1.4736x over previous
import jax
import jax.numpy as jnp
from jax import lax
from jax.experimental import pallas as pl
from jax.experimental.pallas import tpu as pltpu
from jax.experimental.pallas import tpu_sc as plsc

B, T, D = 8, 512, 512
MEL_MAX = 4096

NC, NS = 2, 16
NW = NC * NS
FRAMES_PER_W = B * MEL_MAX // NW
CHUNK = 64
NCHUNK = FRAMES_PER_W // CHUNK
GROUPS = FRAMES_PER_W // 16
ZERO_ROW = B * T


def _lr_body(x_hbm, tgt_hbm, out_hbm, tgt_v, cums_v, gidx_v, zbuf, buf0, buf1,
             zsem, sem0, sem1):
    cid = lax.axis_index("c")
    sid = lax.axis_index("s")
    wid = sid * NC + cid
    b = wid & 7
    q4 = wid >> 3

    zcp = pltpu.make_async_copy(x_hbm.at[pl.ds(ZERO_ROW, CHUNK)], zbuf, zsem)
    zcp.start()

    pltpu.sync_copy(tgt_hbm.at[b], tgt_v)
    zcp.wait()

    def cum_step(i, carry):
        v = tgt_v[pl.ds(i * 16, 16)]
        cums_v[pl.ds(i * 16, 16)] = plsc.cumsum(v) + carry
        return carry + jnp.sum(v)

    total = lax.fori_loop(0, T // 16, cum_step, jnp.int32(0))

    lane = lax.iota(jnp.int32, 16)

    def bs_step(gi, _):
        ck = gi // 4
        t = gi % 4
        j = (q4 + 4 * ck) * CHUNK + t * 16 + lane
        idx = jnp.zeros((16,), jnp.int32)
        for s in (256, 128, 64, 32, 16, 8, 4, 2, 1):
            val = plsc.load_gather(cums_v, [idx + (s - 1)])
            idx = jnp.where(val <= j, idx + s, idx)
        val = plsc.load_gather(cums_v, [idx])
        cnt = idx + (val <= j).astype(jnp.int32)
        cnt = jnp.minimum(cnt, T - 1)
        g = jnp.where(j < total, b * T + cnt, ZERO_ROW + (j & 63))
        gidx_v[pl.ds(gi * 16, 16)] = g
        return 0

    lax.fori_loop(0, GROUPS, bs_step, 0)

    bufs = (buf0, buf1)
    sems = (sem0, sem1)

    def gcp(c, k):
        return pltpu.make_async_copy(
            x_hbm.at[gidx_v.at[pl.ds(c * CHUNK, CHUNK)]], bufs[k], sems[k])

    def valid(c):
        return (q4 + 4 * c) * CHUNK < total

    def start(c, k):
        @pl.when(valid(c))
        def _():
            gcp(c, k).start()

    start(0, 0)
    start(1, 1)
    for c in range(NCHUNK):
        k = c % 2
        out_slice = out_hbm.at[pl.ds((b * (MEL_MAX // CHUNK) + q4 + 4 * c) * CHUNK,
                                     CHUNK)]

        @pl.when(valid(c))
        def _(c=c, k=k, out_slice=out_slice):
            gcp(c, k).wait()
            pltpu.sync_copy(bufs[k], out_slice)

        @pl.when(jnp.logical_not(valid(c)))
        def _(out_slice=out_slice):
            pltpu.sync_copy(zbuf, out_slice)

        if c + 2 < NCHUNK:
            start(c + 2, k)


def _length_regulate(x_pad, target):
    mesh = plsc.VectorSubcoreMesh(
        core_axis_name="c", subcore_axis_name="s", num_cores=NC, num_subcores=NS)
    k = pl.kernel(
        _lr_body,
        out_type=jax.ShapeDtypeStruct((B * MEL_MAX, D), jnp.float32),
        mesh=mesh,
        compiler_params=pltpu.CompilerParams(needs_layout_passes=False),
        scratch_types=[
            pltpu.VMEM((T,), jnp.int32),
            pltpu.VMEM((T,), jnp.int32),
            pltpu.VMEM((FRAMES_PER_W,), jnp.int32),
            pltpu.VMEM((CHUNK, D), jnp.float32),
            pltpu.VMEM((CHUNK, D), jnp.float32),
            pltpu.VMEM((CHUNK, D), jnp.float32),
            pltpu.SemaphoreType.DMA,
            pltpu.SemaphoreType.DMA,
            pltpu.SemaphoreType.DMA,
        ],
    )
    return k(x_pad, target)


def _dp_body(x_ref, w1_ref, b1_ref, w2_ref, b2_ref, o_ref):
    xb = x_ref[0]
    h = jnp.dot(xb, w1_ref[...], preferred_element_type=jnp.float32) + b1_ref[...]
    h = jnp.maximum(h, 0.0)
    d = jnp.sum(h * w2_ref[...], axis=1, keepdims=True) + b2_ref[0, 0]
    d = jnp.maximum(d, 0.0)
    o_ref[...] = jnp.broadcast_to(d, (T, 128))


def _duration_predictor(x, W1, b1, W2, b2):
    out = pl.pallas_call(
        _dp_body,
        grid=(B,),
        in_specs=[
            pl.BlockSpec((1, T, D), lambda i: (i, 0, 0)),
            pl.BlockSpec((D, D), lambda i: (0, 0)),
            pl.BlockSpec((1, D), lambda i: (0, 0)),
            pl.BlockSpec((1, D), lambda i: (0, 0)),
            pl.BlockSpec((1, 1), lambda i: (0, 0)),
        ],
        out_specs=pl.BlockSpec((T, 128), lambda i: (i, 0)),
        out_shape=jax.ShapeDtypeStruct((B * T, 128), jnp.float32),
    )(x, W1, b1.reshape(1, D), W2.reshape(1, D), b2.reshape(1, 1))
    return out[:, 0].reshape(B, T)


def kernel(x, target, mel_max_length, W1, b1, W2, b2):
    del mel_max_length
    dp = _duration_predictor(x, W1, b1, W2, b2)
    x_pad = jnp.concatenate(
        [x.reshape(B * T, D), jnp.zeros((64, D), x.dtype)], axis=0)
    out = _length_regulate(x_pad, target).reshape(B, MEL_MAX, D)
    return out, dp
